# Initial kernel scaffold; baseline (speedup 1.0000x reference)
import jax
import jax.numpy as jnp
from jax.experimental import pallas as pl


def kernel(input, target):
    raise NotImplementedError("write your pallas kernel here")


# speedup vs baseline: 46.7723x; 46.7723x over previous
import jax
import jax.numpy as jnp
from jax.experimental import pallas as pl
from jax.experimental.pallas import tpu as pltpu

_ROWS = 4096
_COLS = 512
_N = _ROWS * _COLS
_BLK = 512
_GRID = _ROWS // _BLK
_NTHR = 7
_NPASS = 6
_HI0 = 2139095040


def _selection(out_ref, pat_ref, stat_ref):
    pos_sum = stat_ref[0]
    n_pos = stat_ref[1]
    n_neg = _N - n_pos
    k = jnp.minimum(n_neg, jnp.floor(3.0 * n_pos))
    k = jnp.maximum(k, 1.0)

    def pass_body(_, carry):
        lo, hi, c_lo, c_hi = carry
        step = jax.lax.shift_right_logical(hi - lo, 3)
        thrs = [lo + step * (j + 1) for j in range(_NTHR)]

        def chunk_body(c, cs):
            blk = pat_ref[pl.ds(c * _BLK, _BLK), :]
            return tuple(
                cs[j] + jnp.sum((blk > thrs[j]).astype(jnp.float32))
                for j in range(_NTHR)
            )

        cnts = jax.lax.fori_loop(0, _GRID, chunk_body, (0.0,) * _NTHR)
        q = sum((cnts[j] >= k).astype(jnp.int32) for j in range(_NTHR))
        new_lo = lo + step * q
        new_hi = jnp.where(q == _NTHR, hi, lo + step * (q + 1))
        new_c_lo = c_lo
        new_c_hi = c_hi
        for j in range(_NTHR):
            new_c_lo = jnp.where(q == j + 1, cnts[j], new_c_lo)
            new_c_hi = jnp.where(q == j, cnts[j], new_c_hi)
        return new_lo, new_hi, new_c_lo, new_c_hi

    lo, hi, c_lo, c_hi = jax.lax.fori_loop(
        0, _NPASS, pass_body,
        (jnp.int32(-1), jnp.int32(_HI0), n_neg, jnp.float32(0.0)),
    )

    def sum_body(c, carry):
        s_hi, s_lo = carry
        blk = pat_ref[pl.ds(c * _BLK, _BLK), :]
        v = jax.lax.bitcast_convert_type(blk, jnp.float32)
        s_hi = s_hi + jnp.sum(jnp.where(blk > hi, v, 0.0))
        s_lo = s_lo + jnp.sum(jnp.where(blk > lo, v, 0.0))
        return s_hi, s_lo

    s_hi, s_lo = jax.lax.fori_loop(0, _GRID, sum_body, (0.0, 0.0))
    m = k - c_hi
    c_bin = jnp.maximum(c_lo - c_hi, 1.0)
    topk = s_hi + m * (s_lo - s_hi) / c_bin
    denom = jnp.maximum(n_pos + k, 1.0)
    out_ref[0, 0] = (pos_sum + topk) / denom


def _ohem_body(x_ref, t_ref, out_ref, pat_ref, stat_ref):
    i = pl.program_id(0)
    x = x_ref[...]
    t = t_ref[...]
    loss = (jnp.maximum(x, 0.0) - x * t
            + jnp.log(1.0 + jnp.exp(-jnp.abs(x))))
    pos = t > 0.5
    pat = jnp.where(pos, jnp.int32(-1),
                    jax.lax.bitcast_convert_type(loss, jnp.int32))
    pat_ref[pl.ds(i * _BLK, _BLK), :] = pat
    ps = jnp.sum(jnp.where(pos, loss, 0.0))
    npos = jnp.sum(pos.astype(jnp.float32))

    @pl.when(i == 0)
    def _():
        stat_ref[0] = ps
        stat_ref[1] = npos

    @pl.when(i != 0)
    def _():
        stat_ref[0] = stat_ref[0] + ps
        stat_ref[1] = stat_ref[1] + npos

    @pl.when(i == _GRID - 1)
    def _():
        _selection(out_ref, pat_ref, stat_ref)


def kernel(input, target):
    x = input.reshape(_ROWS, _COLS)
    t = target.reshape(_ROWS, _COLS)
    out = pl.pallas_call(
        _ohem_body,
        grid=(_GRID,),
        in_specs=[
            pl.BlockSpec((_BLK, _COLS), lambda i: (i, 0)),
            pl.BlockSpec((_BLK, _COLS), lambda i: (i, 0)),
        ],
        out_specs=pl.BlockSpec(memory_space=pltpu.SMEM),
        out_shape=jax.ShapeDtypeStruct((1, 1), jnp.float32),
        scratch_shapes=[
            pltpu.VMEM((_ROWS, _COLS), jnp.int32),
            pltpu.SMEM((2,), jnp.float32),
        ],
    )(x, t)
    return out[0, 0]


# speedup vs baseline: 61.7003x; 1.3192x over previous
import jax
import jax.numpy as jnp
from jax.experimental import pallas as pl
from jax.experimental.pallas import tpu as pltpu

_ROWS = 4096
_COLS = 512
_N = _ROWS * _COLS
_BLK = 512
_GRID = _ROWS // _BLK
_NTHR = 7
_NPASS = 6
_HI0 = 2139095040


def _selection(out_ref, pat_ref, acc_ref):
    pos_sum = jnp.sum(acc_ref[0])
    n_pos = jnp.sum(acc_ref[1])
    n_neg = _N - n_pos
    k = jnp.minimum(n_neg, jnp.floor(3.0 * n_pos))
    k = jnp.maximum(k, 1.0)

    zeros = jnp.zeros((8, _COLS), jnp.float32)

    def pass_body(_, carry):
        lo, hi, c_lo, c_hi = carry
        step = jax.lax.shift_right_logical(hi - lo, 3)
        thrs = [lo + step * (j + 1) for j in range(_NTHR)]

        def chunk_body(c, accs):
            blk = pat_ref[pl.ds(c * _BLK, _BLK), :].reshape(_BLK // 8, 8, _COLS)
            return tuple(
                accs[j] + jnp.sum((blk > thrs[j]).astype(jnp.float32), axis=0)
                for j in range(_NTHR)
            )

        accs = jax.lax.fori_loop(0, _GRID, chunk_body, (zeros,) * _NTHR)
        cnts = [jnp.sum(accs[j]) for j in range(_NTHR)]
        q = sum((cnts[j] >= k).astype(jnp.int32) for j in range(_NTHR))
        new_lo = lo + step * q
        new_hi = jnp.where(q == _NTHR, hi, lo + step * (q + 1))
        new_c_lo = c_lo
        new_c_hi = c_hi
        for j in range(_NTHR):
            new_c_lo = jnp.where(q == j + 1, cnts[j], new_c_lo)
            new_c_hi = jnp.where(q == j, cnts[j], new_c_hi)
        return new_lo, new_hi, new_c_lo, new_c_hi

    lo, hi, c_lo, c_hi = jax.lax.fori_loop(
        0, _NPASS, pass_body,
        (jnp.int32(-1), jnp.int32(_HI0), n_neg, jnp.float32(0.0)),
    )

    def sum_body(c, carry):
        a_hi, a_lo = carry
        blk = pat_ref[pl.ds(c * _BLK, _BLK), :].reshape(_BLK // 8, 8, _COLS)
        v = jax.lax.bitcast_convert_type(blk, jnp.float32)
        a_hi = a_hi + jnp.sum(jnp.where(blk > hi, v, 0.0), axis=0)
        a_lo = a_lo + jnp.sum(jnp.where(blk > lo, v, 0.0), axis=0)
        return a_hi, a_lo

    a_hi, a_lo = jax.lax.fori_loop(0, _GRID, sum_body, (zeros, zeros))
    s_hi = jnp.sum(a_hi)
    s_lo = jnp.sum(a_lo)
    m = k - c_hi
    c_bin = jnp.maximum(c_lo - c_hi, 1.0)
    topk = s_hi + m * (s_lo - s_hi) / c_bin
    denom = jnp.maximum(n_pos + k, 1.0)
    out_ref[0, 0] = (pos_sum + topk) / denom


def _ohem_body(x_ref, t_ref, out_ref, pat_ref, acc_ref):
    i = pl.program_id(0)
    x = x_ref[...]
    t = t_ref[...]
    loss = (jnp.maximum(x, 0.0) - x * t
            + jnp.log(1.0 + jnp.exp(-jnp.abs(x))))
    pos = t > 0.5
    pat = jnp.where(pos, jnp.int32(-1),
                    jax.lax.bitcast_convert_type(loss, jnp.int32))
    pat_ref[pl.ds(i * _BLK, _BLK), :] = pat
    l3 = loss.reshape(_BLK // 8, 8, _COLS)
    p3 = pos.reshape(_BLK // 8, 8, _COLS)
    ps = jnp.sum(jnp.where(p3, l3, 0.0), axis=0)
    npos = jnp.sum(p3.astype(jnp.float32), axis=0)

    @pl.when(i == 0)
    def _():
        acc_ref[0] = ps
        acc_ref[1] = npos

    @pl.when(i != 0)
    def _():
        acc_ref[0] = acc_ref[0] + ps
        acc_ref[1] = acc_ref[1] + npos

    @pl.when(i == _GRID - 1)
    def _():
        _selection(out_ref, pat_ref, acc_ref)


def kernel(input, target):
    x = input.reshape(_ROWS, _COLS)
    t = target.reshape(_ROWS, _COLS)
    out = pl.pallas_call(
        _ohem_body,
        grid=(_GRID,),
        in_specs=[
            pl.BlockSpec((_BLK, _COLS), lambda i: (i, 0)),
            pl.BlockSpec((_BLK, _COLS), lambda i: (i, 0)),
        ],
        out_specs=pl.BlockSpec(memory_space=pltpu.SMEM),
        out_shape=jax.ShapeDtypeStruct((1, 1), jnp.float32),
        scratch_shapes=[
            pltpu.VMEM((_ROWS, _COLS), jnp.int32),
            pltpu.VMEM((2, 8, _COLS), jnp.float32),
        ],
    )(x, t)
    return out[0, 0]


# speedup vs baseline: 85.3356x; 1.3831x over previous
import jax
import jax.numpy as jnp
from jax.experimental import pallas as pl
from jax.experimental.pallas import tpu as pltpu

_ROWS = 4096
_COLS = 512
_N = _ROWS * _COLS
_BLK = 512
_GRID = _ROWS // _BLK
_NTHR = 3
_NPASS = 8
_HI0 = 2139095040
_STEP0 = (_HI0 + 1) >> 2
_T0 = [-1 + _STEP0 * (j + 1) for j in range(_NTHR)]


def _selection(out_ref, pat_ref, acc_ref):
    pos_sum = jnp.sum(acc_ref[0])
    n_pos = jnp.sum(acc_ref[1])
    cnts0 = [jnp.sum(acc_ref[2 + j]) for j in range(_NTHR)]
    n_neg = _N - n_pos
    k = jnp.minimum(n_neg, jnp.floor(3.0 * n_pos))
    k = jnp.maximum(k, 1.0)

    zeros = jnp.zeros((8, _COLS), jnp.float32)

    def narrow(lo, hi, c_lo, c_hi, step, thrs, cnts):
        q = sum((cnts[j] >= k).astype(jnp.int32) for j in range(_NTHR))
        new_lo = lo + step * q
        new_hi = jnp.where(q == _NTHR, hi, lo + step * (q + 1))
        new_c_lo = c_lo
        new_c_hi = c_hi
        for j in range(_NTHR):
            new_c_lo = jnp.where(q == j + 1, cnts[j], new_c_lo)
            new_c_hi = jnp.where(q == j, cnts[j], new_c_hi)
        return new_lo, new_hi, new_c_lo, new_c_hi

    lo, hi, c_lo, c_hi = narrow(
        jnp.int32(-1), jnp.int32(_HI0), n_neg, jnp.float32(0.0),
        jnp.int32(_STEP0), _T0, cnts0)

    def pass_body(_, carry):
        lo, hi, c_lo, c_hi = carry
        step = jax.lax.shift_right_logical(hi - lo, 2)
        thrs = [lo + step * (j + 1) for j in range(_NTHR)]

        def chunk_body(c, accs):
            blk = pat_ref[pl.ds(c * _BLK, _BLK), :].reshape(_BLK // 8, 8, _COLS)
            return tuple(
                accs[j] + jnp.sum((blk > thrs[j]).astype(jnp.float32), axis=0)
                for j in range(_NTHR)
            )

        accs = jax.lax.fori_loop(0, _GRID, chunk_body, (zeros,) * _NTHR)
        cnts = [jnp.sum(accs[j]) for j in range(_NTHR)]
        return narrow(lo, hi, c_lo, c_hi, step, thrs, cnts)

    lo, hi, c_lo, c_hi = jax.lax.fori_loop(
        0, _NPASS, pass_body, (lo, hi, c_lo, c_hi))

    def sum_body(c, carry):
        a_hi, a_lo = carry
        blk = pat_ref[pl.ds(c * _BLK, _BLK), :].reshape(_BLK // 8, 8, _COLS)
        v = jax.lax.bitcast_convert_type(blk, jnp.float32)
        a_hi = a_hi + jnp.sum(jnp.where(blk > hi, v, 0.0), axis=0)
        a_lo = a_lo + jnp.sum(jnp.where(blk > lo, v, 0.0), axis=0)
        return a_hi, a_lo

    a_hi, a_lo = jax.lax.fori_loop(0, _GRID, sum_body, (zeros, zeros))
    s_hi = jnp.sum(a_hi)
    s_lo = jnp.sum(a_lo)
    m = k - c_hi
    c_bin = jnp.maximum(c_lo - c_hi, 1.0)
    topk = s_hi + m * (s_lo - s_hi) / c_bin
    denom = jnp.maximum(n_pos + k, 1.0)
    out_ref[0, 0] = (pos_sum + topk) / denom


def _ohem_body(x_ref, t_ref, out_ref, pat_ref, acc_ref):
    i = pl.program_id(0)
    x = x_ref[...]
    t = t_ref[...]
    loss = (jnp.maximum(x, 0.0) - x * t
            + jnp.log(1.0 + jnp.exp(-jnp.abs(x))))
    pos = t > 0.5
    pat = jnp.where(pos, jnp.int32(-1),
                    jax.lax.bitcast_convert_type(loss, jnp.int32))
    pat_ref[pl.ds(i * _BLK, _BLK), :] = pat
    l3 = loss.reshape(_BLK // 8, 8, _COLS)
    t3 = t.reshape(_BLK // 8, 8, _COLS)
    p3 = pat.reshape(_BLK // 8, 8, _COLS)
    ps = jnp.sum(l3 * t3, axis=0)
    npos = jnp.sum(t3, axis=0)
    cnt = [jnp.sum((p3 > _T0[j]).astype(jnp.float32), axis=0)
           for j in range(_NTHR)]

    @pl.when(i == 0)
    def _():
        acc_ref[0] = ps
        acc_ref[1] = npos
        for j in range(_NTHR):
            acc_ref[2 + j] = cnt[j]

    @pl.when(i != 0)
    def _():
        acc_ref[0] = acc_ref[0] + ps
        acc_ref[1] = acc_ref[1] + npos
        for j in range(_NTHR):
            acc_ref[2 + j] = acc_ref[2 + j] + cnt[j]

    @pl.when(i == _GRID - 1)
    def _():
        _selection(out_ref, pat_ref, acc_ref)


def kernel(input, target):
    x = input.reshape(_ROWS, _COLS)
    t = target.reshape(_ROWS, _COLS)
    out = pl.pallas_call(
        _ohem_body,
        grid=(_GRID,),
        in_specs=[
            pl.BlockSpec((_BLK, _COLS), lambda i: (i, 0)),
            pl.BlockSpec((_BLK, _COLS), lambda i: (i, 0)),
        ],
        out_specs=pl.BlockSpec(memory_space=pltpu.SMEM),
        out_shape=jax.ShapeDtypeStruct((1, 1), jnp.float32),
        scratch_shapes=[
            pltpu.VMEM((_ROWS, _COLS), jnp.int32),
            pltpu.VMEM((2 + _NTHR, 8, _COLS), jnp.float32),
        ],
    )(x, t)
    return out[0, 0]
